# baseline (device time: 456501 ns/iter reference)
import jax
import jax.numpy as jnp
from jax import lax
from jax.experimental import pallas as pl
from jax.experimental.pallas import tpu as pltpu

M = 16384
N_OUT = 1024
HALF = M // 2
CHUNK = 1024
N_CHUNKS = HALF // CHUNK
KSLOTS = 4


def kernel(x):
    def body(x_hbm, out_ref, commx, keepb, sums, sx, rx, lk, sy, ry, lc):
        i = pl.program_id(0)
        mx = lax.axis_index("x")
        my = lax.axis_index("y")
        peer_x = (1 - mx, my)
        peer_y = (mx, 1 - my)
        slot2 = lax.rem(i, 2)
        prev2 = 1 - slot2
        slot4 = lax.rem(i, 4)
        r0 = my * HALF
        cmine = mx * N_OUT
        cpeer = (1 - mx) * N_OUT

        def x_copy(j, jslot):
            return pltpu.make_async_remote_copy(
                src_ref=x_hbm.at[0, pl.ds(r0 + j * CHUNK, CHUNK), pl.ds(cpeer, N_OUT)],
                dst_ref=commx.at[jslot],
                send_sem=sx.at[jslot],
                recv_sem=rx.at[jslot],
                device_id=peer_x,
                device_id_type=pl.DeviceIdType.MESH,
            )

        def keep_copy(j, jslot):
            return pltpu.make_async_copy(
                x_hbm.at[0, pl.ds(r0 + j * CHUNK, CHUNK), pl.ds(cmine, N_OUT)],
                keepb.at[jslot],
                lk.at[jslot],
            )

        def y_copy(j, sum_slot):
            return pltpu.make_async_remote_copy(
                src_ref=sums.at[sum_slot],
                dst_ref=out_ref.at[pl.ds(r0 + j * CHUNK, CHUNK), :],
                send_sem=sy.at[j],
                recv_sem=ry.at[j],
                device_id=peer_y,
                device_id_type=pl.DeviceIdType.MESH,
            )

        def local_copy(j, jslot):
            return pltpu.make_async_copy(
                sums.at[jslot],
                out_ref.at[pl.ds(r0 + j * CHUNK, CHUNK), :],
                lc.at[jslot],
            )

        @pl.when(i == 0)
        def _():
            barrier = pltpu.get_barrier_semaphore()
            for nbr in (peer_x, peer_y):
                pl.semaphore_signal(
                    barrier,
                    inc=1,
                    device_id=nbr,
                    device_id_type=pl.DeviceIdType.MESH,
                )
            pl.semaphore_wait(barrier, 2)
            for j in range(N_CHUNKS):
                x_copy(j, j).start()
            for j in range(KSLOTS):
                keep_copy(j, j).start()

        x_copy(i, i).wait()
        keep_copy(i, slot4).wait()

        @pl.when(i >= 2)
        def _():
            y_copy(i - 2, slot2).wait_send()
            local_copy(i - 2, slot2).wait()

        sums[slot2] = keepb[slot4] + commx[i]

        y_copy(i, slot2).start()
        local_copy(i, slot2).start()

        @pl.when(i + KSLOTS < N_CHUNKS)
        def _():
            keep_copy(i + KSLOTS, slot4).start()

        @pl.when(i == N_CHUNKS - 1)
        def _():
            y_copy(i, slot2).wait_send()
            y_copy(i - 1, prev2).wait_send()
            local_copy(i, slot2).wait()
            local_copy(i - 1, prev2).wait()
            for j in range(N_CHUNKS):
                y_copy(j, 0).wait_recv()

    return pl.pallas_call(
        body,
        grid=(N_CHUNKS,),
        out_shape=jax.ShapeDtypeStruct((M, N_OUT), jnp.float32),
        in_specs=[pl.BlockSpec(memory_space=pl.ANY)],
        out_specs=pl.BlockSpec(memory_space=pl.ANY),
        scratch_shapes=[
            pltpu.VMEM((N_CHUNKS, CHUNK, N_OUT), jnp.float32),
            pltpu.VMEM((KSLOTS, CHUNK, N_OUT), jnp.float32),
            pltpu.VMEM((2, CHUNK, N_OUT), jnp.float32),
            pltpu.SemaphoreType.DMA((N_CHUNKS,)),
            pltpu.SemaphoreType.DMA((N_CHUNKS,)),
            pltpu.SemaphoreType.DMA((KSLOTS,)),
            pltpu.SemaphoreType.DMA((N_CHUNKS,)),
            pltpu.SemaphoreType.DMA((N_CHUNKS,)),
            pltpu.SemaphoreType.DMA((2,)),
        ],
        compiler_params=pltpu.CompilerParams(
            collective_id=0,
            dimension_semantics=("arbitrary",),
            vmem_limit_bytes=60 * 1024 * 1024,
        ),
    )(x)


# device time: 423143 ns/iter; 1.0788x vs baseline; 1.0788x over previous
import jax
import jax.numpy as jnp
from jax import lax
from jax.experimental import pallas as pl
from jax.experimental.pallas import tpu as pltpu

M = 16384
N_OUT = 1024
HALF = M // 2
CHUNK = 256
N_CHUNKS = HALF // CHUNK
KSLOTS = 4


def kernel(x):
    def body(x_hbm, out_ref, commx, keepb, sums, sx, rx, lk, sy, ry, lc):
        i = pl.program_id(0)
        mx = lax.axis_index("x")
        my = lax.axis_index("y")
        peer_x = (1 - mx, my)
        peer_y = (mx, 1 - my)
        slot2 = lax.rem(i, 2)
        prev2 = 1 - slot2
        slot4 = lax.rem(i, 4)
        r0 = my * HALF
        cmine = mx * N_OUT
        cpeer = (1 - mx) * N_OUT

        def x_copy(j, jslot):
            return pltpu.make_async_remote_copy(
                src_ref=x_hbm.at[0, pl.ds(r0 + j * CHUNK, CHUNK), pl.ds(cpeer, N_OUT)],
                dst_ref=commx.at[jslot],
                send_sem=sx.at[jslot],
                recv_sem=rx.at[jslot],
                device_id=peer_x,
                device_id_type=pl.DeviceIdType.MESH,
            )

        def keep_copy(j, jslot):
            return pltpu.make_async_copy(
                x_hbm.at[0, pl.ds(r0 + j * CHUNK, CHUNK), pl.ds(cmine, N_OUT)],
                keepb.at[jslot],
                lk.at[jslot],
            )

        def y_copy(j, sum_slot):
            return pltpu.make_async_remote_copy(
                src_ref=sums.at[sum_slot],
                dst_ref=out_ref.at[pl.ds(r0 + j * CHUNK, CHUNK), :],
                send_sem=sy.at[j],
                recv_sem=ry.at[j],
                device_id=peer_y,
                device_id_type=pl.DeviceIdType.MESH,
            )

        def local_copy(j, jslot):
            return pltpu.make_async_copy(
                sums.at[jslot],
                out_ref.at[pl.ds(r0 + j * CHUNK, CHUNK), :],
                lc.at[jslot],
            )

        @pl.when(i == 0)
        def _():
            barrier = pltpu.get_barrier_semaphore()
            for nbr in (peer_x, peer_y):
                pl.semaphore_signal(
                    barrier,
                    inc=1,
                    device_id=nbr,
                    device_id_type=pl.DeviceIdType.MESH,
                )
            pl.semaphore_wait(barrier, 2)
            for j in range(N_CHUNKS):
                x_copy(j, j).start()
            for j in range(KSLOTS):
                keep_copy(j, j).start()

        x_copy(i, i).wait()
        keep_copy(i, slot4).wait()

        @pl.when(i >= 2)
        def _():
            y_copy(i - 2, slot2).wait_send()
            local_copy(i - 2, slot2).wait()

        sums[slot2] = keepb[slot4] + commx[i]

        y_copy(i, slot2).start()
        local_copy(i, slot2).start()

        @pl.when(i + KSLOTS < N_CHUNKS)
        def _():
            keep_copy(i + KSLOTS, slot4).start()

        @pl.when(i == N_CHUNKS - 1)
        def _():
            y_copy(i, slot2).wait_send()
            y_copy(i - 1, prev2).wait_send()
            local_copy(i, slot2).wait()
            local_copy(i - 1, prev2).wait()
            for j in range(N_CHUNKS):
                y_copy(j, 0).wait_recv()

    return pl.pallas_call(
        body,
        grid=(N_CHUNKS,),
        out_shape=jax.ShapeDtypeStruct((M, N_OUT), jnp.float32),
        in_specs=[pl.BlockSpec(memory_space=pl.ANY)],
        out_specs=pl.BlockSpec(memory_space=pl.ANY),
        scratch_shapes=[
            pltpu.VMEM((N_CHUNKS, CHUNK, N_OUT), jnp.float32),
            pltpu.VMEM((KSLOTS, CHUNK, N_OUT), jnp.float32),
            pltpu.VMEM((2, CHUNK, N_OUT), jnp.float32),
            pltpu.SemaphoreType.DMA((N_CHUNKS,)),
            pltpu.SemaphoreType.DMA((N_CHUNKS,)),
            pltpu.SemaphoreType.DMA((KSLOTS,)),
            pltpu.SemaphoreType.DMA((N_CHUNKS,)),
            pltpu.SemaphoreType.DMA((N_CHUNKS,)),
            pltpu.SemaphoreType.DMA((2,)),
        ],
        compiler_params=pltpu.CompilerParams(
            collective_id=0,
            dimension_semantics=("arbitrary",),
            vmem_limit_bytes=60 * 1024 * 1024,
        ),
    )(x)
